# baseline (speedup 1.0000x reference)
import jax
import jax.numpy as jnp
from jax import lax
from jax.experimental import pallas as pl
from jax.experimental.pallas import tpu as pltpu

_K = 91
_Q = 900
_N = 64
_B = 32
_THRESH = 0.3


def _body(os_ref, lt_ref, bt_ref, ot_ref):
  x = lt_ref[...]
  m = jnp.max(x, axis=0)
  i = lax.broadcasted_iota(jnp.int32, (_K, _B, _Q), 0)
  a = jnp.min(jnp.where(x == m[None], i, _K), axis=0)
  s = 1.0 / (1.0 + jnp.exp(-m))
  keep = s >= _THRESH

  b = bt_ref[...]
  w_sz = os_ref[0, 1].astype(jnp.float32)
  h_sz = os_ref[0, 0].astype(jnp.float32)
  zero = jnp.zeros((), jnp.float32)
  ot_ref[0] = jnp.where(keep, a.astype(jnp.float32), zero)
  ot_ref[1] = jnp.where(keep, s, zero)
  ot_ref[2] = jnp.where(keep, (b[0] - 0.5 * b[2]) * w_sz, zero)
  ot_ref[3] = jnp.where(keep, (b[1] - 0.5 * b[3]) * h_sz, zero)
  ot_ref[4] = jnp.where(keep, b[2] * w_sz, zero)
  ot_ref[5] = jnp.where(keep, b[3] * h_sz, zero)


@jax.jit
def kernel(logits, boxes, original_sizes):
  n, q, k = logits.shape
  lt = jnp.transpose(logits, (2, 0, 1))
  bt = jnp.transpose(boxes, (2, 0, 1))
  ot = pl.pallas_call(
      _body,
      grid=(n // _B,),
      in_specs=[
          pl.BlockSpec(memory_space=pltpu.SMEM),
          pl.BlockSpec((k, _B, q), lambda i: (0, i, 0)),
          pl.BlockSpec((4, _B, q), lambda i: (0, i, 0)),
      ],
      out_specs=pl.BlockSpec((6, _B, q), lambda i: (0, i, 0)),
      out_shape=jax.ShapeDtypeStruct((6, n, q), jnp.float32),
      compiler_params=pltpu.CompilerParams(
          allow_input_fusion=[False, False, True]),
  )(original_sizes, lt, bt)
  return jnp.transpose(ot, (1, 2, 0))


# speedup vs baseline: 1.1700x; 1.1700x over previous
import jax
import jax.numpy as jnp
from jax import lax
from jax.experimental import pallas as pl
from jax.experimental.pallas import tpu as pltpu

_K = 91
_Q = 900
_N = 64
_B = 16
_THRESH = 0.3


def _body(os_ref, lt_ref, bt_ref, ot_ref):
  x = lt_ref[...]
  m = jnp.max(x, axis=0)
  i = lax.broadcasted_iota(jnp.int32, (_K, _B, _Q), 0)
  a = jnp.min(jnp.where(x == m[None], i, _K), axis=0)
  s = 1.0 / (1.0 + jnp.exp(-m))
  keep = s >= _THRESH

  b = bt_ref[...]
  w_sz = os_ref[0, 1].astype(jnp.float32)
  h_sz = os_ref[0, 0].astype(jnp.float32)
  zero = jnp.zeros((), jnp.float32)
  ot_ref[0] = jnp.where(keep, a.astype(jnp.float32), zero)
  ot_ref[1] = jnp.where(keep, s, zero)
  ot_ref[2] = jnp.where(keep, (b[0] - 0.5 * b[2]) * w_sz, zero)
  ot_ref[3] = jnp.where(keep, (b[1] - 0.5 * b[3]) * h_sz, zero)
  ot_ref[4] = jnp.where(keep, b[2] * w_sz, zero)
  ot_ref[5] = jnp.where(keep, b[3] * h_sz, zero)


@jax.jit
def kernel(logits, boxes, original_sizes):
  n, q, k = logits.shape
  lt = jnp.transpose(logits, (2, 0, 1))
  bt = jnp.transpose(boxes, (2, 0, 1))
  ot = pl.pallas_call(
      _body,
      grid=(n // _B,),
      in_specs=[
          pl.BlockSpec(memory_space=pltpu.SMEM),
          pl.BlockSpec((k, _B, q), lambda i: (0, i, 0)),
          pl.BlockSpec((4, _B, q), lambda i: (0, i, 0)),
      ],
      out_specs=pl.BlockSpec((6, _B, q), lambda i: (0, i, 0)),
      out_shape=jax.ShapeDtypeStruct((6, n, q), jnp.float32),
      compiler_params=pltpu.CompilerParams(
          allow_input_fusion=[False, False, True]),
  )(original_sizes, lt, bt)
  return jnp.transpose(ot, (1, 2, 0))
